# Initial kernel scaffold; baseline (speedup 1.0000x reference)
import functools

import jax
import jax.numpy as jnp
from jax import lax
from jax.experimental import pallas as pl
from jax.experimental.pallas import tpu as pltpu
from jax.experimental.pallas import tpu_sc as plsc

B = 4096
L = 50
D = 64
LANES = 16
SC_CORES = 2
SC_SUBCORES = 16
NW = SC_CORES * SC_SUBCORES
ROWS = 2 * B
ROWS_PER_W = ROWS // NW
CHUNK = 64

_MLP_BM = 512


def _pool_sc(s, w, table):
    mesh = plsc.VectorSubcoreMesh(core_axis_name="c", subcore_axis_name="s")

    @functools.partial(
        pl.kernel,
        out_type=jax.ShapeDtypeStruct((ROWS, D), jnp.float32),
        mesh=mesh,
        scratch_types=[
            pltpu.VMEM((CHUNK, L), jnp.int32),
            pltpu.VMEM((CHUNK, L), jnp.float32),
            pltpu.VMEM((L, D), jnp.float32),
            pltpu.VMEM((CHUNK, D), jnp.float32),
        ],
    )
    def pool_kernel(s_hbm, w_hbm, table_hbm, out_hbm, idx_v, w_v, rows_v, out_v):
        wid = lax.axis_index("s") * SC_CORES + lax.axis_index("c")
        base = wid * ROWS_PER_W

        @pl.loop(0, ROWS_PER_W, step=CHUNK)
        def _chunk(r0):
            pltpu.sync_copy(s_hbm.at[pl.ds(base + r0, CHUNK)], idx_v)
            pltpu.sync_copy(w_hbm.at[pl.ds(base + r0, CHUNK)], w_v)

            @pl.loop(0, CHUNK)
            def _row(r):
                pltpu.sync_copy(table_hbm.at[idx_v.at[r]], rows_v)

                def tbody(t, accs):
                    wt = w_v[r, t]
                    return tuple(
                        acc + wt * rows_v[t, pl.ds(c * LANES, LANES)]
                        for c, acc in enumerate(accs)
                    )

                accs = lax.fori_loop(
                    0, L, tbody,
                    tuple(jnp.zeros((LANES,), jnp.float32) for _ in range(D // LANES)),
                )
                for c in range(D // LANES):
                    out_v[r, pl.ds(c * LANES, LANES)] = accs[c] * (1.0 / L)

            pltpu.sync_copy(out_v, out_hbm.at[pl.ds(base + r0, CHUNK)])

    return pool_kernel(s, w, table)


def _mlp_body(p1_ref, p2_ref, W1_ref, b1_ref, W2_ref, b2_ref, W3_ref, b3_ref,
              out_ref):
    a = p1_ref[...]
    b = p2_ref[...]
    comb = jnp.concatenate([jnp.abs(a - b), a * b], axis=1)
    hp = jax.lax.Precision.HIGHEST
    h = jnp.dot(comb, W1_ref[...], precision=hp,
                preferred_element_type=jnp.float32) + b1_ref[...]
    h = jnp.dot(h, W2_ref[...], precision=hp,
                preferred_element_type=jnp.float32) + b2_ref[...]
    out_ref[...] = jnp.dot(h, W3_ref[...], precision=hp,
                           preferred_element_type=jnp.float32) + b3_ref[...]


def _mlp_tc(pooled, W1, b1, W2, b2, W3, b3):
    nblk = B // _MLP_BM
    h1 = W1.shape[1]
    nc = W3.shape[1]
    return pl.pallas_call(
        _mlp_body,
        grid=(nblk,),
        in_specs=[
            pl.BlockSpec((_MLP_BM, D), lambda i: (i, 0)),
            pl.BlockSpec((_MLP_BM, D), lambda i: (i + nblk, 0)),
            pl.BlockSpec((2 * D, h1), lambda i: (0, 0)),
            pl.BlockSpec((1, h1), lambda i: (0, 0)),
            pl.BlockSpec((h1, h1), lambda i: (0, 0)),
            pl.BlockSpec((1, h1), lambda i: (0, 0)),
            pl.BlockSpec((h1, nc), lambda i: (0, 0)),
            pl.BlockSpec((1, nc), lambda i: (0, 0)),
        ],
        out_specs=pl.BlockSpec((_MLP_BM, nc), lambda i: (i, 0)),
        out_shape=jax.ShapeDtypeStruct((B, nc), jnp.float32),
    )(pooled, pooled, W1, b1[None, :], W2, b2[None, :], W3, b3[None, :])


def kernel(s1, s2, w1, w2, table, W1, b1, W2, b2, W3, b3):
    s = jnp.concatenate([s1, s2], axis=0).astype(jnp.int32)
    w = jnp.concatenate([w1, w2], axis=0)
    pooled = _pool_sc(s, w, table)
    return _mlp_tc(pooled, W1, b1, W2, b2, W3, b3)


# speedup vs baseline: 1.2271x; 1.2271x over previous
import functools

import jax
import jax.numpy as jnp
from jax import lax
from jax.experimental import pallas as pl
from jax.experimental.pallas import tpu as pltpu
from jax.experimental.pallas import tpu_sc as plsc

B = 4096
L = 50
D = 64
LANES = 16
SC_CORES = 2
SC_SUBCORES = 16
NW = SC_CORES * SC_SUBCORES
ROWS = 2 * B
ROWS_PER_W = ROWS // NW
CHUNK = 64
LW = 64

_MLP_BM = 512


def _pool_sc(s, w, table):
    mesh = plsc.VectorSubcoreMesh(core_axis_name="c", subcore_axis_name="s")

    @functools.partial(
        pl.kernel,
        out_type=jax.ShapeDtypeStruct((ROWS, D), jnp.float32),
        mesh=mesh,
        scratch_types=[
            pltpu.VMEM((CHUNK, L), jnp.int32),
            pltpu.VMEM((CHUNK, LW), jnp.float32),
            pltpu.VMEM((L, D), jnp.float32),
            pltpu.VMEM((CHUNK, D), jnp.float32),
        ],
        compiler_params=pltpu.CompilerParams(use_tc_tiling_on_sc=False),
    )
    def pool_kernel(s_hbm, w_hbm, table_hbm, out_hbm, idx_v, w_v, rows_v, out_v):
        wid = lax.axis_index("s") * SC_CORES + lax.axis_index("c")
        base = wid * ROWS_PER_W

        @pl.loop(0, ROWS_PER_W, step=CHUNK)
        def _chunk(r0):
            pltpu.sync_copy(s_hbm.at[pl.ds(base + r0, CHUNK)], idx_v)
            pltpu.sync_copy(w_hbm.at[pl.ds(base + r0, CHUNK)], w_v)

            @pl.loop(0, CHUNK)
            def _row(r):
                pltpu.sync_copy(table_hbm.at[idx_v.at[r]], rows_v)

                accs = [jnp.zeros((LANES,), jnp.float32)
                        for _ in range(D // LANES)]
                for g in range((L + LANES - 1) // LANES):
                    wvec = w_v[r, pl.ds(g * LANES, LANES)]
                    for j in range(min(LANES, L - g * LANES)):
                        t = g * LANES + j
                        wt = wvec[j]
                        for c in range(D // LANES):
                            accs[c] = accs[c] + wt * rows_v[t, pl.ds(c * LANES, LANES)]
                for c in range(D // LANES):
                    out_v[r, pl.ds(c * LANES, LANES)] = accs[c] * (1.0 / L)

            pltpu.sync_copy(out_v, out_hbm.at[pl.ds(base + r0, CHUNK)])

    return pool_kernel(s, w, table)


def _mlp_body(p1_ref, p2_ref, W1_ref, b1_ref, W2_ref, b2_ref, W3_ref, b3_ref,
              out_ref):
    a = p1_ref[...]
    b = p2_ref[...]
    comb = jnp.concatenate([jnp.abs(a - b), a * b], axis=1)
    hp = jax.lax.Precision.HIGHEST
    h = jnp.dot(comb, W1_ref[...], precision=hp,
                preferred_element_type=jnp.float32) + b1_ref[...]
    h = jnp.dot(h, W2_ref[...], precision=hp,
                preferred_element_type=jnp.float32) + b2_ref[...]
    out_ref[...] = jnp.dot(h, W3_ref[...], precision=hp,
                           preferred_element_type=jnp.float32) + b3_ref[...]


def _mlp_tc(pooled, W1, b1, W2, b2, W3, b3):
    nblk = B // _MLP_BM
    h1 = W1.shape[1]
    nc = W3.shape[1]
    return pl.pallas_call(
        _mlp_body,
        grid=(nblk,),
        in_specs=[
            pl.BlockSpec((_MLP_BM, D), lambda i: (i, 0)),
            pl.BlockSpec((_MLP_BM, D), lambda i: (i + nblk, 0)),
            pl.BlockSpec((2 * D, h1), lambda i: (0, 0)),
            pl.BlockSpec((1, h1), lambda i: (0, 0)),
            pl.BlockSpec((h1, h1), lambda i: (0, 0)),
            pl.BlockSpec((1, h1), lambda i: (0, 0)),
            pl.BlockSpec((h1, nc), lambda i: (0, 0)),
            pl.BlockSpec((1, nc), lambda i: (0, 0)),
        ],
        out_specs=pl.BlockSpec((_MLP_BM, nc), lambda i: (i, 0)),
        out_shape=jax.ShapeDtypeStruct((B, nc), jnp.float32),
    )(pooled, pooled, W1, b1[None, :], W2, b2[None, :], W3, b3[None, :])


def kernel(s1, s2, w1, w2, table, W1, b1, W2, b2, W3, b3):
    s = jnp.concatenate([s1, s2], axis=0).astype(jnp.int32)
    w = jnp.concatenate([w1, w2], axis=0)
    w = jnp.pad(w, ((0, 0), (0, LW - L)))
    pooled = _pool_sc(s, w, table)
    return _mlp_tc(pooled, W1, b1, W2, b2, W3, b3)


# speedup vs baseline: 1.2383x; 1.0091x over previous
import functools

import jax
import jax.numpy as jnp
from jax import lax
from jax.experimental import pallas as pl
from jax.experimental.pallas import tpu as pltpu
from jax.experimental.pallas import tpu_sc as plsc

B = 4096
L = 50
D = 64
LANES = 16
SC_CORES = 2
SC_SUBCORES = 16
NW = SC_CORES * SC_SUBCORES
ROWS = 2 * B
ROWS_PER_W = ROWS // NW
CHUNK = 64

_MLP_BM = 512


_W_GROUPS = [(0, 0, LANES), (16, 0, LANES), (32, 0, LANES), (34, 14, LANES)]


def _pool_sc(s1, s2, w1, w2, table):
    mesh = plsc.VectorSubcoreMesh(core_axis_name="c", subcore_axis_name="s")
    half = NW // 2
    rows_per_w = B // half

    @functools.partial(
        pl.kernel,
        out_type=jax.ShapeDtypeStruct((ROWS, D), jnp.float32),
        mesh=mesh,
        scratch_types=[
            pltpu.VMEM((CHUNK, L), jnp.int32),
            pltpu.VMEM((CHUNK, L), jnp.float32),
            pltpu.VMEM((L, D), jnp.float32),
            pltpu.VMEM((CHUNK, D), jnp.float32),
        ],
        compiler_params=pltpu.CompilerParams(use_tc_tiling_on_sc=False),
    )
    def pool_kernel(s1_hbm, s2_hbm, w1_hbm, w2_hbm, table_hbm, out_hbm,
                    idx_v, w_v, rows_v, out_v):
        wid = lax.axis_index("s") * SC_CORES + lax.axis_index("c")

        def do_half(s_hbm, w_hbm, lwid, out_base):
            lbase = lwid * rows_per_w

            @pl.loop(0, rows_per_w, step=CHUNK)
            def _chunk(r0):
                pltpu.sync_copy(s_hbm.at[pl.ds(lbase + r0, CHUNK)], idx_v)
                pltpu.sync_copy(w_hbm.at[pl.ds(lbase + r0, CHUNK)], w_v)

                @pl.loop(0, CHUNK)
                def _row(r):
                    pltpu.sync_copy(table_hbm.at[idx_v.at[r]], rows_v)

                    accs = [jnp.zeros((LANES,), jnp.float32)
                            for _ in range(D // LANES)]
                    for off, j0, j1 in _W_GROUPS:
                        wvec = w_v[r, pl.ds(off, LANES)]
                        for j in range(j0, j1):
                            t = off + j
                            wt = wvec[j]
                            for c in range(D // LANES):
                                accs[c] = accs[c] + wt * rows_v[t, pl.ds(c * LANES, LANES)]
                    for c in range(D // LANES):
                        out_v[r, pl.ds(c * LANES, LANES)] = accs[c] * (1.0 / L)

                pltpu.sync_copy(out_v, out_hbm.at[pl.ds(out_base + lbase + r0, CHUNK)])

        @pl.when(wid < half)
        def _():
            do_half(s1_hbm, w1_hbm, wid, 0)

        @pl.when(wid >= half)
        def _():
            do_half(s2_hbm, w2_hbm, wid - half, B)

    return pool_kernel(s1, s2, w1, w2, table)


def _mlp_body(p1_ref, p2_ref, W1_ref, b1_ref, W2_ref, b2_ref, W3_ref, b3_ref,
              out_ref):
    a = p1_ref[...]
    b = p2_ref[...]
    comb = jnp.concatenate([jnp.abs(a - b), a * b], axis=1)
    hp = jax.lax.Precision.HIGHEST
    h = jnp.dot(comb, W1_ref[...], precision=hp,
                preferred_element_type=jnp.float32) + b1_ref[...]
    h = jnp.dot(h, W2_ref[...], precision=hp,
                preferred_element_type=jnp.float32) + b2_ref[...]
    out_ref[...] = jnp.dot(h, W3_ref[...], precision=hp,
                           preferred_element_type=jnp.float32) + b3_ref[...]


def _mlp_tc(pooled, W1, b1, W2, b2, W3, b3):
    nblk = B // _MLP_BM
    h1 = W1.shape[1]
    nc = W3.shape[1]
    return pl.pallas_call(
        _mlp_body,
        grid=(nblk,),
        in_specs=[
            pl.BlockSpec((_MLP_BM, D), lambda i: (i, 0)),
            pl.BlockSpec((_MLP_BM, D), lambda i: (i + nblk, 0)),
            pl.BlockSpec((2 * D, h1), lambda i: (0, 0)),
            pl.BlockSpec((1, h1), lambda i: (0, 0)),
            pl.BlockSpec((h1, h1), lambda i: (0, 0)),
            pl.BlockSpec((1, h1), lambda i: (0, 0)),
            pl.BlockSpec((h1, nc), lambda i: (0, 0)),
            pl.BlockSpec((1, nc), lambda i: (0, 0)),
        ],
        out_specs=pl.BlockSpec((_MLP_BM, nc), lambda i: (i, 0)),
        out_shape=jax.ShapeDtypeStruct((B, nc), jnp.float32),
    )(pooled, pooled, W1, b1[None, :], W2, b2[None, :], W3, b3[None, :])


def kernel(s1, s2, w1, w2, table, W1, b1, W2, b2, W3, b3):
    pooled = _pool_sc(s1.astype(jnp.int32), s2.astype(jnp.int32), w1, w2, table)
    return _mlp_tc(pooled, W1, b1, W2, b2, W3, b3)
